# Initial kernel scaffold; baseline (speedup 1.0000x reference)
import jax
import jax.numpy as jnp
from jax.experimental import pallas as pl


def kernel(tokens, embedding_table):
    raise NotImplementedError("write your pallas kernel here")


# speedup vs baseline: 5.4881x; 5.4881x over previous
import functools
import math

import jax
import jax.numpy as jnp
import numpy as np
from jax import lax
from jax.experimental import pallas as pl
from jax.experimental.pallas import tpu as pltpu
from jax.experimental.pallas import tpu_sc as plsc

D = 128
SCALE = np.float32(math.sqrt(float(D)))

NC, NS = 2, 16
NW = NC * NS
CH = 128


def _scale_body(t_ref, o_ref):
    o_ref[...] = t_ref[...] * SCALE


def _scale_table(table):
    v, d = table.shape
    rows = 1000
    assert v % rows == 0
    return pl.pallas_call(
        _scale_body,
        grid=(v // rows,),
        in_specs=[pl.BlockSpec((rows, d), lambda i: (i, 0))],
        out_specs=pl.BlockSpec((rows, d), lambda i: (i, 0)),
        out_shape=jax.ShapeDtypeStruct((v, d), jnp.float32),
    )(table)


def _make_gather(n_chunks):
    rpw = n_chunks * CH
    mesh = plsc.VectorSubcoreMesh(
        core_axis_name="c", subcore_axis_name="s", num_cores=NC, num_subcores=NS
    )

    @functools.partial(
        pl.kernel,
        out_type=jax.ShapeDtypeStruct((NW * rpw, D), jnp.float32),
        mesh=mesh,
        scratch_types=[
            pltpu.VMEM((n_chunks, CH), jnp.int32),
            pltpu.VMEM((CH, D), jnp.float32),
            pltpu.SemaphoreType.DMA,
        ],
    )
    def _gather(table_hbm, idx_hbm, out_hbm, idx_v, rows_v, gsem):
        wid = lax.axis_index("s") * NC + lax.axis_index("c")
        base = wid * rpw
        pltpu.sync_copy(idx_hbm.at[wid], idx_v)

        @pl.loop(0, n_chunks)
        def _chunk(c):
            pltpu.async_copy(table_hbm.at[idx_v.at[c]], rows_v, gsem).wait()
            pltpu.sync_copy(rows_v, out_hbm.at[pl.ds(base + c * CH, CH)])

    return _gather


def kernel(tokens, embedding_table):
    b0, b1 = tokens.shape
    n_tok = b0 * b1
    assert n_tok % (NW * CH) == 0
    n_chunks = n_tok // (NW * CH)
    scaled = _scale_table(embedding_table)
    idx = tokens.reshape(NW, n_chunks, CH).astype(jnp.int32)
    out = _make_gather(n_chunks)(scaled, idx)
    return out.reshape(b0, b1, D)


# speedup vs baseline: 7.5515x; 1.3760x over previous
import functools
import math

import jax
import jax.numpy as jnp
import numpy as np
from jax import lax
from jax.experimental import pallas as pl
from jax.experimental.pallas import tpu as pltpu
from jax.experimental.pallas import tpu_sc as plsc

D = 128
SCALE = np.float32(math.sqrt(float(D)))

NC, NS = 2, 16
NW = NC * NS
CH = 128


def _scale_body(t_ref, o_ref):
    o_ref[...] = t_ref[...] * SCALE


def _scale_table(table):
    v, d = table.shape
    rows = 1000
    assert v % rows == 0
    return pl.pallas_call(
        _scale_body,
        grid=(v // rows,),
        in_specs=[pl.BlockSpec((rows, d), lambda i: (i, 0))],
        out_specs=pl.BlockSpec((rows, d), lambda i: (i, 0)),
        out_shape=jax.ShapeDtypeStruct((v, d), jnp.float32),
    )(table)


def _make_gather(n_chunks):
    rpw = n_chunks * CH
    mesh = plsc.VectorSubcoreMesh(
        core_axis_name="c", subcore_axis_name="s", num_cores=NC, num_subcores=NS
    )

    assert n_chunks % 2 == 0 and n_chunks >= 4

    @functools.partial(
        pl.kernel,
        out_type=jax.ShapeDtypeStruct((NW * rpw, D), jnp.float32),
        mesh=mesh,
        scratch_types=[
            pltpu.VMEM((n_chunks, CH), jnp.int32),
            pltpu.VMEM((2, CH, D), jnp.float32),
            pltpu.SemaphoreType.DMA,
            pltpu.SemaphoreType.DMA,
            pltpu.SemaphoreType.DMA,
            pltpu.SemaphoreType.DMA,
        ],
    )
    def _gather(table_hbm, idx_hbm, out_hbm, idx_v, rows_v, g0, g1, s0, s1):
        wid = lax.axis_index("s") * NC + lax.axis_index("c")
        base = wid * rpw
        gsem = (g0, g1)
        ssem = (s0, s1)
        pltpu.sync_copy(idx_hbm.at[wid], idx_v)

        def gather_start(b, c):
            pltpu.async_copy(table_hbm.at[idx_v.at[c]], rows_v.at[b], gsem[b])

        def gather_wait(b, c):
            pltpu.make_async_copy(
                table_hbm.at[idx_v.at[c]], rows_v.at[b], gsem[b]
            ).wait()

        def scatter_start(b, c):
            pltpu.async_copy(
                rows_v.at[b], out_hbm.at[pl.ds(base + c * CH, CH)], ssem[b]
            )

        def scatter_wait(b, c):
            pltpu.make_async_copy(
                rows_v.at[b], out_hbm.at[pl.ds(base + c * CH, CH)], ssem[b]
            ).wait()

        gather_start(0, 0)
        gather_start(1, 1)

        @pl.loop(0, n_chunks // 2 - 1)
        def _pair(g):
            for b in range(2):
                c = 2 * g + b
                gather_wait(b, c)
                scatter_start(b, c)
                scatter_wait(b, c)
                gather_start(b, c + 2)

        for b in range(2):
            c = n_chunks - 2 + b
            gather_wait(b, c)
            scatter_start(b, c)
            scatter_wait(b, c)

    return _gather


def kernel(tokens, embedding_table):
    b0, b1 = tokens.shape
    n_tok = b0 * b1
    assert n_tok % (NW * CH) == 0
    n_chunks = n_tok // (NW * CH)
    scaled = _scale_table(embedding_table)
    idx = tokens.reshape(NW, n_chunks, CH).astype(jnp.int32)
    out = _make_gather(n_chunks)(scaled, idx)
    return out.reshape(b0, b1, D)


# speedup vs baseline: 8.3017x; 1.0993x over previous
import functools
import math

import jax
import jax.numpy as jnp
import numpy as np
from jax import lax
from jax.experimental import pallas as pl
from jax.experimental.pallas import tpu as pltpu
from jax.experimental.pallas import tpu_sc as plsc

D = 128
SCALE = np.float32(math.sqrt(float(D)))

NC, NS = 2, 16
NW = NC * NS
CH = 128


def _scale_body(t_ref, o_ref):
    o_ref[...] = t_ref[...] * SCALE


def _scale_table(table):
    v, d = table.shape
    rows = 4000
    assert v % rows == 0
    return pl.pallas_call(
        _scale_body,
        grid=(v // rows,),
        in_specs=[pl.BlockSpec((rows, d), lambda i: (i, 0))],
        out_specs=pl.BlockSpec((rows, d), lambda i: (i, 0)),
        out_shape=jax.ShapeDtypeStruct((v, d), jnp.float32),
    )(table)


def _make_gather(n_chunks):
    rpw = n_chunks * CH
    mesh = plsc.VectorSubcoreMesh(
        core_axis_name="c", subcore_axis_name="s", num_cores=NC, num_subcores=NS
    )

    assert n_chunks % 2 == 0 and n_chunks >= 4

    @functools.partial(
        pl.kernel,
        out_type=jax.ShapeDtypeStruct((NW * rpw, D), jnp.float32),
        mesh=mesh,
        scratch_types=[
            pltpu.VMEM((n_chunks, CH), jnp.int32),
            pltpu.VMEM((2, CH, D), jnp.float32),
            pltpu.SemaphoreType.DMA,
            pltpu.SemaphoreType.DMA,
            pltpu.SemaphoreType.DMA,
            pltpu.SemaphoreType.DMA,
        ],
    )
    def _gather(table_hbm, idx_hbm, out_hbm, idx_v, rows_v, g0, g1, s0, s1):
        wid = lax.axis_index("s") * NC + lax.axis_index("c")
        base = wid * rpw
        gsem = (g0, g1)
        ssem = (s0, s1)
        pltpu.sync_copy(idx_hbm.at[wid], idx_v)

        def gather_start(b, c):
            pltpu.async_copy(table_hbm.at[idx_v.at[c]], rows_v.at[b], gsem[b])

        def gather_wait(b, c):
            pltpu.make_async_copy(
                table_hbm.at[idx_v.at[c]], rows_v.at[b], gsem[b]
            ).wait()

        def scatter_start(b, c):
            pltpu.async_copy(
                rows_v.at[b], out_hbm.at[pl.ds(base + c * CH, CH)], ssem[b]
            )

        def scatter_wait(b, c):
            pltpu.make_async_copy(
                rows_v.at[b], out_hbm.at[pl.ds(base + c * CH, CH)], ssem[b]
            ).wait()

        gather_start(0, 0)
        gather_start(1, 1)

        @pl.loop(0, n_chunks // 2 - 1)
        def _pair(g):
            for b in range(2):
                c = 2 * g + b
                gather_wait(b, c)
                scatter_start(b, c)
                scatter_wait(b, c)
                gather_start(b, c + 2)

        for b in range(2):
            c = n_chunks - 2 + b
            gather_wait(b, c)
            scatter_start(b, c)
            scatter_wait(b, c)

    return _gather


def kernel(tokens, embedding_table):
    b0, b1 = tokens.shape
    n_tok = b0 * b1
    assert n_tok % (NW * CH) == 0
    n_chunks = n_tok // (NW * CH)
    scaled = _scale_table(embedding_table)
    idx = tokens.reshape(NW, n_chunks, CH).astype(jnp.int32)
    out = _make_gather(n_chunks)(scaled, idx)
    return out.reshape(b0, b1, D)
